# baseline (speedup 1.0000x reference)
import functools

import jax
import jax.numpy as jnp
from jax import lax
from jax.experimental import pallas as pl
from jax.experimental.pallas import tpu as pltpu
from jax.experimental.pallas import tpu_sc as plsc

N = 10000
EMB = 128
NBLK = 10
BLK = N // NBLK
NC = 2
NS = 16
CH = 128
PADN = 10240
RPT = PADN // NS
CW = 128


def _lk(v):
    return jnp.maximum(v, 0.01 * v)


def _dot(a, b):
    return jnp.dot(a, b, preferred_element_type=jnp.float32)


def _embed_body(des_ref, tw_ref, nc_ref, wd, wt, wnc, wid, wit, winc,
                bd, bt, bnc, bi, out_ref):
    d = _lk(_dot(des_ref[...], wd[...]) + bd[...])
    t = _lk(_dot(tw_ref[...], wt[...]) + bt[...])
    c = _lk(_dot(nc_ref[...], wnc[...]) + bnc[...])
    out_ref[...] = _lk(_dot(d, wid[...]) + _dot(t, wit[...])
                       + _dot(c, winc[...]) + bi[...])


def _combine_body(h_ref, s0_ref, s1_ref, c0_ref, c1_ref,
                  root, w0, w1, b, out_ref):
    m0 = s0_ref[0] / jnp.maximum(c0_ref[0][:, 0:1], 1.0)
    m1 = s1_ref[0] / jnp.maximum(c1_ref[0][:, 0:1], 1.0)
    out_ref[...] = (_dot(h_ref[...], root[...]) + _dot(m0, w0[...])
                    + _dot(m1, w1[...]) + b[...])


def _final_body(h_ref, s0_ref, s1_ref, c0_ref, c1_ref,
                root, w0, w1, b, wo1, bo1, wo2, bo2, out_ref):
    m0 = s0_ref[0] / jnp.maximum(c0_ref[0][:, 0:1], 1.0)
    m1 = s1_ref[0] / jnp.maximum(c1_ref[0][:, 0:1], 1.0)
    h3 = (_dot(h_ref[...], root[...]) + _dot(m0, w0[...])
          + _dot(m1, w1[...]) + b[...])
    g = _lk(_dot(h3, wo1[...]) + bo1[...])
    out_ref[...] = _dot(g, wo2[...]) + bo2[...]


def _full(shape):
    return pl.BlockSpec(shape, lambda i: tuple(0 for _ in shape))


def _rows(width):
    return pl.BlockSpec((BLK, width), lambda i: (i, 0))


def _seg_spec(r, width):
    return pl.BlockSpec((1, BLK, width), lambda i, _r=r: (_r, i, 0))


_embed_call = pl.pallas_call(
    _embed_body,
    grid=(NBLK,),
    in_specs=[
        _rows(768), _rows(768), _rows(32),
        _full((768, 32)), _full((768, 32)), _full((32, 64)),
        _full((32, EMB)), _full((32, EMB)), _full((64, EMB)),
        _full((1, 32)), _full((1, 32)), _full((1, 64)), _full((1, EMB)),
    ],
    out_specs=_rows(EMB),
    out_shape=jax.ShapeDtypeStruct((N, EMB), jnp.float32),
)

_combine_call = pl.pallas_call(
    _combine_body,
    grid=(NBLK,),
    in_specs=[
        _rows(EMB), _seg_spec(0, EMB), _seg_spec(1, EMB),
        _seg_spec(0, CW), _seg_spec(1, CW),
        _full((EMB, EMB)), _full((EMB, EMB)), _full((EMB, EMB)),
        _full((1, EMB)),
    ],
    out_specs=_rows(EMB),
    out_shape=jax.ShapeDtypeStruct((N, EMB), jnp.float32),
)

_final_call = pl.pallas_call(
    _final_body,
    grid=(NBLK,),
    in_specs=[
        _rows(EMB), _seg_spec(0, EMB), _seg_spec(1, EMB),
        _seg_spec(0, CW), _seg_spec(1, CW),
        _full((EMB, EMB)), _full((EMB, EMB)), _full((EMB, EMB)),
        _full((1, EMB)),
        _full((EMB, EMB)), _full((1, EMB)), _full((EMB, 2)), _full((1, 2)),
    ],
    out_specs=_rows(2),
    out_shape=jax.ShapeDtypeStruct((N, 2), jnp.float32),
)


def _make_segsum(nchunk):
    assert nchunk % 4 == 0 and nchunk >= 8

    @functools.partial(
        pl.kernel,
        mesh=plsc.VectorSubcoreMesh(core_axis_name="c", subcore_axis_name="s"),
        out_type=jax.ShapeDtypeStruct((NC, PADN, EMB), jnp.float32),
        scratch_types=[
            pltpu.VMEM((nchunk, CH), jnp.int32),
            pltpu.VMEM((2, 2, CH), jnp.int32),
            pltpu.VMEM((2, CH, EMB), jnp.float32),
            pltpu.VMEM_SHARED((PADN, EMB), jnp.float32),
            pltpu.SemaphoreType.DMA,
            pltpu.SemaphoreType.DMA,
        ],
    )
    def segsum(h_hbm, eidx_hbm, out_hbm,
               packed_v, ibuf, rows, acc_sh, sg0, sg1):
        c = lax.axis_index("c")
        s = lax.axis_index("s")
        sem_g = (sg0, sg1)

        def zrow(i, carry):
            for k in range(EMB // 16):
                rows[0, i, pl.ds(k * 16, 16)] = jnp.zeros((16,), jnp.float32)
            return carry
        lax.fori_loop(0, CH, zrow, 0)

        def zcp(t, carry):
            pltpu.sync_copy(rows.at[0],
                            acc_sh.at[pl.ds(s * RPT + t * CH, CH)])
            return carry
        lax.fori_loop(0, RPT // CH, zcp, 0)

        pltpu.sync_copy(eidx_hbm.at[c, s], packed_v)
        plsc.subcore_barrier()

        def unpack(j, p):
            def u(k, carry):
                v = packed_v[j, pl.ds(k * 16, 16)]
                ibuf[p, 0, pl.ds(k * 16, 16)] = lax.shift_right_logical(v, 16)
                ibuf[p, 1, pl.ds(k * 16, 16)] = lax.bitwise_and(v, 0xFFFF)
                return carry
            lax.fori_loop(0, CH // 16, u, 0)

        def gather(p):
            pltpu.async_copy(h_hbm.at[ibuf.at[p, 0]], rows.at[p], sem_g[p])

        def wait_gather(p):
            pltpu.make_async_copy(
                h_hbm.at[ibuf.at[p, 0]], rows.at[p], sem_g[p]).wait()

        def step(j, p, next_gather, unpack2, jnext):
            pltpu.async_copy(h_hbm.at[ibuf.at[p, 0]], rows.at[p],
                             sem_g[p]).wait()
            pltpu.sync_copy(rows.at[p], acc_sh.at[ibuf.at[p, 1]], add=True)
            if unpack2:
                unpack(jnext, p)

        unpack(0, 0)
        unpack(1, 1)

        def body(i, carry):
            base = 4 * i
            for u in range(4):
                step(base + u, u % 2, True, True, base + u + 2)
            return carry
        lax.fori_loop(0, (nchunk - 4) // 4, body, 0)

        for j in range(nchunk - 4, nchunk):
            step(j, j % 2, j + 1 < nchunk, j + 2 < nchunk, j + 2)

        plsc.subcore_barrier()
        pltpu.sync_copy(acc_sh.at[pl.ds(s * RPT, RPT)],
                        out_hbm.at[c, pl.ds(s * RPT, RPT)])

    return segsum


def _make_counts(nchunk):
    @functools.partial(
        pl.kernel,
        mesh=plsc.VectorSubcoreMesh(core_axis_name="c", subcore_axis_name="s"),
        out_type=jax.ShapeDtypeStruct((NC, PADN, CW), jnp.float32),
        scratch_types=[
            pltpu.VMEM((nchunk, CH), jnp.int32),
            pltpu.VMEM((CH, CW), jnp.float32),
            pltpu.VMEM_SHARED((PADN, CW), jnp.float32),
        ],
    )
    def counts(dst_hbm, out_hbm, dst_v, ones_v, acc_sh):
        c = lax.axis_index("c")
        s = lax.axis_index("s")

        def fill(val):
            def body(i, carry):
                for k in range(CW // 16):
                    ones_v[i, pl.ds(k * 16, 16)] = jnp.full(
                        (16,), val, jnp.float32)
                return carry
            lax.fori_loop(0, CH, body, 0)

        fill(0.0)

        def zcp(t, carry):
            pltpu.sync_copy(ones_v, acc_sh.at[pl.ds(s * RPT + t * CH, CH)])
            return carry
        lax.fori_loop(0, RPT // CH, zcp, 0)

        fill(1.0)
        pltpu.sync_copy(dst_hbm.at[c, s], dst_v)
        plsc.subcore_barrier()

        def step(j, carry):
            pltpu.sync_copy(ones_v, acc_sh.at[dst_v.at[j]], add=True)
            return carry
        lax.fori_loop(0, nchunk, step, 0)

        plsc.subcore_barrier()
        pltpu.sync_copy(acc_sh.at[pl.ds(s * RPT, RPT)],
                        out_hbm.at[c, pl.ds(s * RPT, RPT)])

    return counts


def kernel(x, edge_index_follow, edge_index_friend,
           W_des, b_des, W_tweet, b_tweet, W_num, b_num, W_cat, b_cat,
           W_in, b_in, rgcn_weight, rgcn_root, rgcn_bias,
           W_out1, b_out1, W_out2, b_out2):
    E = edge_index_follow.shape[1]
    grp = NS * CH * 4
    ept = 4 * CH * ((E + grp - 1) // grp)
    nchunk = ept // CH
    pade = NS * ept

    des = x[:, 17:785]
    tweet = x[:, 785:1553]
    ncp = jnp.pad(x[:, 0:17], ((0, 0), (0, 15)))

    w_nc = jnp.zeros((32, 64), jnp.float32)
    w_nc = w_nc.at[0:6, 0:32].set(W_num).at[6:17, 32:64].set(W_cat)
    b_nc = jnp.concatenate([b_num, b_cat]).reshape(1, 64)

    def pad_edges(ei):
        src = jnp.concatenate(
            [ei[0], jnp.zeros((pade - E,), jnp.int32)])
        dst = jnp.concatenate(
            [ei[1], jnp.full((pade - E,), N, jnp.int32)])
        return src, dst

    src_f, dst_f = pad_edges(edge_index_follow)
    src_r, dst_r = pad_edges(edge_index_friend)
    src_all = jnp.stack([src_f, src_r]).reshape(NC, NS, nchunk, CH)
    dst_all = jnp.stack([dst_f, dst_r]).reshape(NC, NS, nchunk, CH)
    eidx_all = (src_all << 16) | dst_all

    segsum = _make_segsum(nchunk)
    cnts = _make_counts(nchunk)(dst_all)

    h1 = _embed_call(
        des, tweet, ncp,
        W_des, W_tweet, w_nc,
        W_in[0:32], W_in[32:64], W_in[64:128],
        b_des.reshape(1, 32), b_tweet.reshape(1, 32), b_nc,
        b_in.reshape(1, EMB))

    seg1 = segsum(h1, eidx_all)
    h2 = _combine_call(h1, seg1, seg1, cnts, cnts,
                       rgcn_root, rgcn_weight[0], rgcn_weight[1],
                       rgcn_bias.reshape(1, EMB))

    seg2 = segsum(h2, eidx_all)
    out = _final_call(h2, seg2, seg2, cnts, cnts,
                      rgcn_root, rgcn_weight[0], rgcn_weight[1],
                      rgcn_bias.reshape(1, EMB),
                      W_out1, b_out1.reshape(1, EMB),
                      W_out2, b_out2.reshape(1, 2))
    return out


# speedup vs baseline: 1.0013x; 1.0013x over previous
import functools

import jax
import jax.numpy as jnp
from jax import lax
from jax.experimental import pallas as pl
from jax.experimental.pallas import tpu as pltpu
from jax.experimental.pallas import tpu_sc as plsc

N = 10000
EMB = 128
NBLK = 10
BLK = N // NBLK
NC = 2
NS = 16
CH = 128
PADN = 10240
RPT = PADN // NS
CW = 128


def _lk(v):
    return jnp.maximum(v, 0.01 * v)


def _dot(a, b):
    return jnp.dot(a, b, preferred_element_type=jnp.float32)


def _embed_body(des_ref, tw_ref, nc_ref, wd, wt, wnc, wid, wit, winc,
                bd, bt, bnc, bi, out_ref):
    d = _lk(_dot(des_ref[...], wd[...]) + bd[...])
    t = _lk(_dot(tw_ref[...], wt[...]) + bt[...])
    c = _lk(_dot(nc_ref[...], wnc[...]) + bnc[...])
    out_ref[...] = _lk(_dot(d, wid[...]) + _dot(t, wit[...])
                       + _dot(c, winc[...]) + bi[...])


def _combine_body(h_ref, s0_ref, s1_ref, c0_ref, c1_ref,
                  root, w0, w1, b, out_ref):
    m0 = s0_ref[0] / jnp.maximum(c0_ref[0][:, 0:1], 1.0)
    m1 = s1_ref[0] / jnp.maximum(c1_ref[0][:, 0:1], 1.0)
    out_ref[...] = (_dot(h_ref[...], root[...]) + _dot(m0, w0[...])
                    + _dot(m1, w1[...]) + b[...])


def _final_body(h_ref, s0_ref, s1_ref, c0_ref, c1_ref,
                root, w0, w1, b, wo1, bo1, wo2, bo2, out_ref):
    m0 = s0_ref[0] / jnp.maximum(c0_ref[0][:, 0:1], 1.0)
    m1 = s1_ref[0] / jnp.maximum(c1_ref[0][:, 0:1], 1.0)
    h3 = (_dot(h_ref[...], root[...]) + _dot(m0, w0[...])
          + _dot(m1, w1[...]) + b[...])
    g = _lk(_dot(h3, wo1[...]) + bo1[...])
    out_ref[...] = _dot(g, wo2[...]) + bo2[...]


def _full(shape):
    return pl.BlockSpec(shape, lambda i: tuple(0 for _ in shape))


def _rows(width):
    return pl.BlockSpec((BLK, width), lambda i: (i, 0))


def _seg_spec(r, width):
    return pl.BlockSpec((1, BLK, width), lambda i, _r=r: (_r, i, 0))


_embed_call = pl.pallas_call(
    _embed_body,
    grid=(NBLK,),
    in_specs=[
        _rows(768), _rows(768), _rows(32),
        _full((768, 32)), _full((768, 32)), _full((32, 64)),
        _full((32, EMB)), _full((32, EMB)), _full((64, EMB)),
        _full((1, 32)), _full((1, 32)), _full((1, 64)), _full((1, EMB)),
    ],
    out_specs=_rows(EMB),
    out_shape=jax.ShapeDtypeStruct((N, EMB), jnp.float32),
)

_combine_call = pl.pallas_call(
    _combine_body,
    grid=(NBLK,),
    in_specs=[
        _rows(EMB), _seg_spec(0, EMB), _seg_spec(1, EMB),
        _seg_spec(0, CW), _seg_spec(1, CW),
        _full((EMB, EMB)), _full((EMB, EMB)), _full((EMB, EMB)),
        _full((1, EMB)),
    ],
    out_specs=_rows(EMB),
    out_shape=jax.ShapeDtypeStruct((N, EMB), jnp.float32),
)

_final_call = pl.pallas_call(
    _final_body,
    grid=(NBLK,),
    in_specs=[
        _rows(EMB), _seg_spec(0, EMB), _seg_spec(1, EMB),
        _seg_spec(0, CW), _seg_spec(1, CW),
        _full((EMB, EMB)), _full((EMB, EMB)), _full((EMB, EMB)),
        _full((1, EMB)),
        _full((EMB, EMB)), _full((1, EMB)), _full((EMB, 2)), _full((1, 2)),
    ],
    out_specs=_rows(2),
    out_shape=jax.ShapeDtypeStruct((N, 2), jnp.float32),
)


def _make_segsum(nchunk):
    @functools.partial(
        pl.kernel,
        mesh=plsc.VectorSubcoreMesh(core_axis_name="c", subcore_axis_name="s"),
        out_type=jax.ShapeDtypeStruct((NC, PADN, EMB), jnp.float32),
        scratch_types=[
            pltpu.VMEM((nchunk, CH), jnp.int32),
            pltpu.VMEM((nchunk, CH), jnp.int32),
            pltpu.VMEM((CH, EMB), jnp.float32),
            pltpu.VMEM_SHARED((PADN, EMB), jnp.float32),
            pltpu.SemaphoreType.DMA,
        ],
    )
    def segsum(h_hbm, src_hbm, dst_hbm, out_hbm,
               src_v, dst_v, rows_v, acc_sh, sem):
        c = lax.axis_index("c")
        s = lax.axis_index("s")

        def zrow(i, carry):
            for k in range(EMB // 16):
                rows_v[i, pl.ds(k * 16, 16)] = jnp.zeros((16,), jnp.float32)
            return carry
        lax.fori_loop(0, CH, zrow, 0)

        def zcp(t, carry):
            pltpu.sync_copy(rows_v, acc_sh.at[pl.ds(s * RPT + t * CH, CH)])
            return carry
        lax.fori_loop(0, RPT // CH, zcp, 0)

        pltpu.sync_copy(src_hbm.at[c, s], src_v)
        pltpu.sync_copy(dst_hbm.at[c, s], dst_v)
        plsc.subcore_barrier()

        def step(j, carry):
            pltpu.async_copy(h_hbm.at[src_v.at[j]], rows_v, sem).wait()
            pltpu.sync_copy(rows_v, acc_sh.at[dst_v.at[j]], add=True)
            return carry
        lax.fori_loop(0, nchunk, step, 0)

        plsc.subcore_barrier()
        pltpu.sync_copy(acc_sh.at[pl.ds(s * RPT, RPT)],
                        out_hbm.at[c, pl.ds(s * RPT, RPT)])

    return segsum


def _make_counts(nchunk):
    @functools.partial(
        pl.kernel,
        mesh=plsc.VectorSubcoreMesh(core_axis_name="c", subcore_axis_name="s"),
        out_type=jax.ShapeDtypeStruct((NC, PADN, CW), jnp.float32),
        scratch_types=[
            pltpu.VMEM((nchunk, CH), jnp.int32),
            pltpu.VMEM((CH, CW), jnp.float32),
            pltpu.VMEM_SHARED((PADN, CW), jnp.float32),
        ],
    )
    def counts(dst_hbm, out_hbm, dst_v, ones_v, acc_sh):
        c = lax.axis_index("c")
        s = lax.axis_index("s")

        def fill(val):
            def body(i, carry):
                for k in range(CW // 16):
                    ones_v[i, pl.ds(k * 16, 16)] = jnp.full(
                        (16,), val, jnp.float32)
                return carry
            lax.fori_loop(0, CH, body, 0)

        fill(0.0)

        def zcp(t, carry):
            pltpu.sync_copy(ones_v, acc_sh.at[pl.ds(s * RPT + t * CH, CH)])
            return carry
        lax.fori_loop(0, RPT // CH, zcp, 0)

        fill(1.0)
        pltpu.sync_copy(dst_hbm.at[c, s], dst_v)
        plsc.subcore_barrier()

        def step(j, carry):
            pltpu.sync_copy(ones_v, acc_sh.at[dst_v.at[j]], add=True)
            return carry
        lax.fori_loop(0, nchunk, step, 0)

        plsc.subcore_barrier()
        pltpu.sync_copy(acc_sh.at[pl.ds(s * RPT, RPT)],
                        out_hbm.at[c, pl.ds(s * RPT, RPT)])

    return counts


def kernel(x, edge_index_follow, edge_index_friend,
           W_des, b_des, W_tweet, b_tweet, W_num, b_num, W_cat, b_cat,
           W_in, b_in, rgcn_weight, rgcn_root, rgcn_bias,
           W_out1, b_out1, W_out2, b_out2):
    E = edge_index_follow.shape[1]
    grp = NS * CH * 4
    ept = 4 * CH * ((E + grp - 1) // grp)
    nchunk = ept // CH
    pade = NS * ept

    des = x[:, 17:785]
    tweet = x[:, 785:1553]
    ncp = jnp.pad(x[:, 0:17], ((0, 0), (0, 15)))

    w_nc = jnp.zeros((32, 64), jnp.float32)
    w_nc = w_nc.at[0:6, 0:32].set(W_num).at[6:17, 32:64].set(W_cat)
    b_nc = jnp.concatenate([b_num, b_cat]).reshape(1, 64)

    def pad_edges(ei):
        src = jnp.concatenate(
            [ei[0], jnp.zeros((pade - E,), jnp.int32)])
        dst = jnp.concatenate(
            [ei[1], jnp.full((pade - E,), N, jnp.int32)])
        return src, dst

    src_f, dst_f = pad_edges(edge_index_follow)
    src_r, dst_r = pad_edges(edge_index_friend)
    src_all = jnp.stack([src_f, src_r]).reshape(NC, NS, nchunk, CH)
    dst_all = jnp.stack([dst_f, dst_r]).reshape(NC, NS, nchunk, CH)

    segsum = _make_segsum(nchunk)
    cnts = _make_counts(nchunk)(dst_all)

    h1 = _embed_call(
        des, tweet, ncp,
        W_des, W_tweet, w_nc,
        W_in[0:32], W_in[32:64], W_in[64:128],
        b_des.reshape(1, 32), b_tweet.reshape(1, 32), b_nc,
        b_in.reshape(1, EMB))

    seg1 = segsum(h1, src_all, dst_all)
    h2 = _combine_call(h1, seg1, seg1, cnts, cnts,
                       rgcn_root, rgcn_weight[0], rgcn_weight[1],
                       rgcn_bias.reshape(1, EMB))

    seg2 = segsum(h2, src_all, dst_all)
    out = _final_call(h2, seg2, seg2, cnts, cnts,
                      rgcn_root, rgcn_weight[0], rgcn_weight[1],
                      rgcn_bias.reshape(1, EMB),
                      W_out1, b_out1.reshape(1, EMB),
                      W_out2, b_out2.reshape(1, 2))
    return out


# speedup vs baseline: 1.8062x; 1.8038x over previous
import functools

import jax
import jax.numpy as jnp
from jax import lax
from jax.experimental import pallas as pl
from jax.experimental.pallas import tpu as pltpu
from jax.experimental.pallas import tpu_sc as plsc

N = 10000
EMB = 128
NBLK = 10
BLK = N // NBLK
NC = 2
NS = 16
CH = 128
PADN = 10240
RPT = PADN // NS
CW = 128


def _lk(v):
    return jnp.maximum(v, 0.01 * v)


def _dot(a, b):
    return jnp.dot(a, b, preferred_element_type=jnp.float32)


def _embed_body(des_ref, tw_ref, nc_ref, wd, wt, wnc, wid, wit, winc,
                bd, bt, bnc, bi, out_ref):
    d = _lk(_dot(des_ref[...], wd[...]) + bd[...])
    t = _lk(_dot(tw_ref[...], wt[...]) + bt[...])
    c = _lk(_dot(nc_ref[...], wnc[...]) + bnc[...])
    out_ref[...] = _lk(_dot(d, wid[...]) + _dot(t, wit[...])
                       + _dot(c, winc[...]) + bi[...])


def _combine_body(h_ref, s0_ref, s1_ref, c0_ref, c1_ref,
                  root, w0, w1, b, out_ref):
    m0 = s0_ref[0] / jnp.maximum(c0_ref[0][:, 0:1], 1.0)
    m1 = s1_ref[0] / jnp.maximum(c1_ref[0][:, 0:1], 1.0)
    out_ref[...] = (_dot(h_ref[...], root[...]) + _dot(m0, w0[...])
                    + _dot(m1, w1[...]) + b[...])


def _final_body(h_ref, s0_ref, s1_ref, c0_ref, c1_ref,
                root, w0, w1, b, wo1, bo1, wo2, bo2, out_ref):
    m0 = s0_ref[0] / jnp.maximum(c0_ref[0][:, 0:1], 1.0)
    m1 = s1_ref[0] / jnp.maximum(c1_ref[0][:, 0:1], 1.0)
    h3 = (_dot(h_ref[...], root[...]) + _dot(m0, w0[...])
          + _dot(m1, w1[...]) + b[...])
    g = _lk(_dot(h3, wo1[...]) + bo1[...])
    out_ref[...] = _dot(g, wo2[...]) + bo2[...]


def _full(shape):
    return pl.BlockSpec(shape, lambda i: tuple(0 for _ in shape))


def _rows(width):
    return pl.BlockSpec((BLK, width), lambda i: (i, 0))


def _seg_spec(r, width):
    return pl.BlockSpec((1, BLK, width), lambda i, _r=r: (_r, i, 0))


_embed_call = pl.pallas_call(
    _embed_body,
    grid=(NBLK,),
    in_specs=[
        _rows(768), _rows(768), _rows(32),
        _full((768, 32)), _full((768, 32)), _full((32, 64)),
        _full((32, EMB)), _full((32, EMB)), _full((64, EMB)),
        _full((1, 32)), _full((1, 32)), _full((1, 64)), _full((1, EMB)),
    ],
    out_specs=_rows(EMB),
    out_shape=jax.ShapeDtypeStruct((N, EMB), jnp.float32),
)

_combine_call = pl.pallas_call(
    _combine_body,
    grid=(NBLK,),
    in_specs=[
        _rows(EMB), _seg_spec(0, EMB), _seg_spec(1, EMB),
        _seg_spec(0, CW), _seg_spec(1, CW),
        _full((EMB, EMB)), _full((EMB, EMB)), _full((EMB, EMB)),
        _full((1, EMB)),
    ],
    out_specs=_rows(EMB),
    out_shape=jax.ShapeDtypeStruct((N, EMB), jnp.float32),
)

_final_call = pl.pallas_call(
    _final_body,
    grid=(NBLK,),
    in_specs=[
        _rows(EMB), _seg_spec(0, EMB), _seg_spec(1, EMB),
        _seg_spec(0, CW), _seg_spec(1, CW),
        _full((EMB, EMB)), _full((EMB, EMB)), _full((EMB, EMB)),
        _full((1, EMB)),
        _full((EMB, EMB)), _full((1, EMB)), _full((EMB, 2)), _full((1, 2)),
    ],
    out_specs=_rows(2),
    out_shape=jax.ShapeDtypeStruct((N, 2), jnp.float32),
)


def _make_segsum(nchunk):
    @functools.partial(
        pl.kernel,
        mesh=plsc.VectorSubcoreMesh(core_axis_name="c", subcore_axis_name="s"),
        out_type=jax.ShapeDtypeStruct((NC, PADN, EMB), jnp.float32),
        scratch_types=[
            pltpu.VMEM((nchunk, CH), jnp.int32),
            pltpu.VMEM((nchunk, CH), jnp.int32),
            pltpu.VMEM((CH, EMB), jnp.float32),
            pltpu.VMEM_SHARED((PADN, EMB), jnp.float32),
            pltpu.SemaphoreType.DMA,
        ],
    )
    def segsum(h_hbm, src_hbm, dst_hbm, out_hbm,
               src_v, dst_v, rows_v, acc_sh, sem):
        c = lax.axis_index("c")
        s = lax.axis_index("s")

        def zrow(i, carry):
            for k in range(EMB // 16):
                rows_v[i, pl.ds(k * 16, 16)] = jnp.zeros((16,), jnp.float32)
            return carry
        lax.fori_loop(0, CH, zrow, 0)

        def zcp(t, carry):
            pltpu.sync_copy(rows_v, acc_sh.at[pl.ds(s * RPT + t * CH, CH)])
            return carry
        lax.fori_loop(0, RPT // CH, zcp, 0)

        pltpu.sync_copy(src_hbm.at[c, s], src_v)
        pltpu.sync_copy(dst_hbm.at[c, s], dst_v)
        plsc.subcore_barrier()

        def step(j, carry):
            pltpu.async_copy(h_hbm.at[src_v.at[j]], rows_v, sem).wait()
            pltpu.sync_copy(rows_v, acc_sh.at[dst_v.at[j]], add=True)
            return carry
        lax.fori_loop(0, nchunk, step, 0)

        plsc.subcore_barrier()
        pltpu.sync_copy(acc_sh.at[pl.ds(s * RPT, RPT)],
                        out_hbm.at[c, pl.ds(s * RPT, RPT)])

    return segsum


def _make_counts(nchunk):
    @functools.partial(
        pl.kernel,
        mesh=plsc.VectorSubcoreMesh(core_axis_name="c", subcore_axis_name="s"),
        out_type=jax.ShapeDtypeStruct((NC, PADN, CW), jnp.float32),
        scratch_types=[
            pltpu.VMEM((nchunk, CH), jnp.int32),
            pltpu.VMEM((CH, CW), jnp.float32),
            pltpu.VMEM_SHARED((PADN, CW), jnp.float32),
        ],
    )
    def counts(dst_hbm, out_hbm, dst_v, ones_v, acc_sh):
        c = lax.axis_index("c")
        s = lax.axis_index("s")

        def fill(val):
            def body(i, carry):
                for k in range(CW // 16):
                    ones_v[i, pl.ds(k * 16, 16)] = jnp.full(
                        (16,), val, jnp.float32)
                return carry
            lax.fori_loop(0, CH, body, 0)

        fill(0.0)

        def zcp(t, carry):
            pltpu.sync_copy(ones_v, acc_sh.at[pl.ds(s * RPT + t * CH, CH)])
            return carry
        lax.fori_loop(0, RPT // CH, zcp, 0)

        fill(1.0)
        pltpu.sync_copy(dst_hbm.at[c, s], dst_v)
        plsc.subcore_barrier()

        def step(j, carry):
            pltpu.sync_copy(ones_v, acc_sh.at[dst_v.at[j]], add=True)
            return carry
        lax.fori_loop(0, nchunk, step, 0)

        plsc.subcore_barrier()
        pltpu.sync_copy(acc_sh.at[pl.ds(s * RPT, RPT)],
                        out_hbm.at[c, pl.ds(s * RPT, RPT)])

    return counts


def kernel(x, edge_index_follow, edge_index_friend,
           W_des, b_des, W_tweet, b_tweet, W_num, b_num, W_cat, b_cat,
           W_in, b_in, rgcn_weight, rgcn_root, rgcn_bias,
           W_out1, b_out1, W_out2, b_out2):
    E = edge_index_follow.shape[1]
    grp = NS * CH * 4
    ept = 4 * CH * ((E + grp - 1) // grp)
    nchunk = ept // CH
    pade = NS * ept

    des = x[:, 17:785]
    tweet = x[:, 785:1553]
    ncp = jnp.pad(x[:, 0:17], ((0, 0), (0, 15)))

    w_nc = jnp.zeros((32, 64), jnp.float32)
    w_nc = w_nc.at[0:6, 0:32].set(W_num).at[6:17, 32:64].set(W_cat)
    b_nc = jnp.concatenate([b_num, b_cat]).reshape(1, 64)

    pad_dst = N + (jnp.arange(pade - E, dtype=jnp.int32) % (PADN - N))
    pad_src = jnp.arange(pade - E, dtype=jnp.int32) % N

    def pad_edges(ei):
        src = jnp.concatenate([ei[0], pad_src])
        dst = jnp.concatenate([ei[1], pad_dst])
        return src, dst

    src_f, dst_f = pad_edges(edge_index_follow)
    src_r, dst_r = pad_edges(edge_index_friend)
    src_all = jnp.stack([src_f, src_r]).reshape(NC, NS, nchunk, CH)
    dst_all = jnp.stack([dst_f, dst_r]).reshape(NC, NS, nchunk, CH)

    segsum = _make_segsum(nchunk)
    cnts = _make_counts(nchunk)(dst_all)

    h1 = _embed_call(
        des, tweet, ncp,
        W_des, W_tweet, w_nc,
        W_in[0:32], W_in[32:64], W_in[64:128],
        b_des.reshape(1, 32), b_tweet.reshape(1, 32), b_nc,
        b_in.reshape(1, EMB))

    seg1 = segsum(h1, src_all, dst_all)
    h2 = _combine_call(h1, seg1, seg1, cnts, cnts,
                       rgcn_root, rgcn_weight[0], rgcn_weight[1],
                       rgcn_bias.reshape(1, EMB))

    seg2 = segsum(h2, src_all, dst_all)
    out = _final_call(h2, seg2, seg2, cnts, cnts,
                      rgcn_root, rgcn_weight[0], rgcn_weight[1],
                      rgcn_bias.reshape(1, EMB),
                      W_out1, b_out1.reshape(1, EMB),
                      W_out2, b_out2.reshape(1, 2))
    return out


# speedup vs baseline: 2.3234x; 1.2864x over previous
import functools

import jax
import jax.numpy as jnp
from jax import lax
from jax.experimental import pallas as pl
from jax.experimental.pallas import tpu as pltpu
from jax.experimental.pallas import tpu_sc as plsc

N = 10000
EMB = 128
NBLK = 10
BLK = N // NBLK
NC = 2
NS = 16
CH = 128
PADN = 10240
RPT = PADN // NS
CW = 128


def _lk(v):
    return jnp.maximum(v, 0.01 * v)


def _dot(a, b):
    return jnp.dot(a, b, preferred_element_type=jnp.float32)


def _embed_body(des_ref, tw_ref, nc_ref, wd, wt, wnc, wid, wit, winc,
                bd, bt, bnc, bi, out_ref):
    d = _lk(_dot(des_ref[...], wd[...]) + bd[...])
    t = _lk(_dot(tw_ref[...], wt[...]) + bt[...])
    c = _lk(_dot(nc_ref[...], wnc[...]) + bnc[...])
    out_ref[...] = _lk(_dot(d, wid[...]) + _dot(t, wit[...])
                       + _dot(c, winc[...]) + bi[...])


def _combine_body(h_ref, s0_ref, s1_ref, c0_ref, c1_ref,
                  root, w0, w1, b, out_ref):
    m0 = s0_ref[0] / jnp.maximum(c0_ref[0][:, 0:1], 1.0)
    m1 = s1_ref[0] / jnp.maximum(c1_ref[0][:, 0:1], 1.0)
    out_ref[...] = (_dot(h_ref[...], root[...]) + _dot(m0, w0[...])
                    + _dot(m1, w1[...]) + b[...])


def _final_body(h_ref, s0_ref, s1_ref, c0_ref, c1_ref,
                root, w0, w1, b, wo1, bo1, wo2, bo2, out_ref):
    m0 = s0_ref[0] / jnp.maximum(c0_ref[0][:, 0:1], 1.0)
    m1 = s1_ref[0] / jnp.maximum(c1_ref[0][:, 0:1], 1.0)
    h3 = (_dot(h_ref[...], root[...]) + _dot(m0, w0[...])
          + _dot(m1, w1[...]) + b[...])
    g = _lk(_dot(h3, wo1[...]) + bo1[...])
    out_ref[...] = _dot(g, wo2[...]) + bo2[...]


def _full(shape):
    return pl.BlockSpec(shape, lambda i: tuple(0 for _ in shape))


def _rows(width):
    return pl.BlockSpec((BLK, width), lambda i: (i, 0))


def _seg_spec(r, width):
    return pl.BlockSpec((1, BLK, width), lambda i, _r=r: (_r, i, 0))


_embed_call = pl.pallas_call(
    _embed_body,
    grid=(NBLK,),
    in_specs=[
        _rows(768), _rows(768), _rows(32),
        _full((768, 32)), _full((768, 32)), _full((32, 64)),
        _full((32, EMB)), _full((32, EMB)), _full((64, EMB)),
        _full((1, 32)), _full((1, 32)), _full((1, 64)), _full((1, EMB)),
    ],
    out_specs=_rows(EMB),
    out_shape=jax.ShapeDtypeStruct((N, EMB), jnp.float32),
)

_combine_call = pl.pallas_call(
    _combine_body,
    grid=(NBLK,),
    in_specs=[
        _rows(EMB), _seg_spec(0, EMB), _seg_spec(1, EMB),
        _seg_spec(0, CW), _seg_spec(1, CW),
        _full((EMB, EMB)), _full((EMB, EMB)), _full((EMB, EMB)),
        _full((1, EMB)),
    ],
    out_specs=_rows(EMB),
    out_shape=jax.ShapeDtypeStruct((N, EMB), jnp.float32),
)

_final_call = pl.pallas_call(
    _final_body,
    grid=(NBLK,),
    in_specs=[
        _rows(EMB), _seg_spec(0, EMB), _seg_spec(1, EMB),
        _seg_spec(0, CW), _seg_spec(1, CW),
        _full((EMB, EMB)), _full((EMB, EMB)), _full((EMB, EMB)),
        _full((1, EMB)),
        _full((EMB, EMB)), _full((1, EMB)), _full((EMB, 2)), _full((1, 2)),
    ],
    out_specs=_rows(2),
    out_shape=jax.ShapeDtypeStruct((N, 2), jnp.float32),
)


def _make_segsum(nchunk):
    assert nchunk % 4 == 0 and nchunk >= 8

    @functools.partial(
        pl.kernel,
        mesh=plsc.VectorSubcoreMesh(core_axis_name="c", subcore_axis_name="s"),
        out_type=jax.ShapeDtypeStruct((NC, PADN, EMB), jnp.float32),
        scratch_types=[
            pltpu.VMEM((nchunk, CH), jnp.int32),
            pltpu.VMEM((2, 2, CH), jnp.int32),
            pltpu.VMEM((2, CH, EMB), jnp.float32),
            pltpu.VMEM_SHARED((PADN, EMB), jnp.float32),
            pltpu.SemaphoreType.DMA,
            pltpu.SemaphoreType.DMA,
        ],
    )
    def segsum(h_hbm, eidx_hbm, out_hbm,
               packed_v, ibuf, rows, acc_sh, sg0, sg1):
        c = lax.axis_index("c")
        s = lax.axis_index("s")
        sem_g = (sg0, sg1)

        def zrow(i, carry):
            for k in range(EMB // 16):
                rows[0, i, pl.ds(k * 16, 16)] = jnp.zeros((16,), jnp.float32)
            return carry
        lax.fori_loop(0, CH, zrow, 0)

        def zcp(t, carry):
            pltpu.sync_copy(rows.at[0],
                            acc_sh.at[pl.ds(s * RPT + t * CH, CH)])
            return carry
        lax.fori_loop(0, RPT // CH, zcp, 0)

        pltpu.sync_copy(eidx_hbm.at[c, s], packed_v)
        plsc.subcore_barrier()

        def unpack(j, p):
            def u(k, carry):
                v = packed_v[j, pl.ds(k * 16, 16)]
                ibuf[p, 0, pl.ds(k * 16, 16)] = lax.shift_right_logical(v, 16)
                ibuf[p, 1, pl.ds(k * 16, 16)] = lax.bitwise_and(v, 0xFFFF)
                return carry
            lax.fori_loop(0, CH // 16, u, 0)

        def gather(p):
            pltpu.async_copy(h_hbm.at[ibuf.at[p, 0]], rows.at[p], sem_g[p])

        def wait_gather(p):
            pltpu.make_async_copy(
                h_hbm.at[ibuf.at[p, 0]], rows.at[p], sem_g[p]).wait()

        def step(j, p, next_gather, unpack2, jnext):
            q = 1 - p
            if next_gather:
                gather(q)
            wait_gather(p)
            pltpu.sync_copy(rows.at[p], acc_sh.at[ibuf.at[p, 1]], add=True)
            if unpack2:
                unpack(jnext, p)

        unpack(0, 0)
        gather(0)
        unpack(1, 1)

        def body(i, carry):
            base = 4 * i
            for u in range(4):
                step(base + u, u % 2, True, True, base + u + 2)
            return carry
        lax.fori_loop(0, (nchunk - 4) // 4, body, 0)

        for j in range(nchunk - 4, nchunk):
            step(j, j % 2, j + 1 < nchunk, j + 2 < nchunk, j + 2)

        plsc.subcore_barrier()
        pltpu.sync_copy(acc_sh.at[pl.ds(s * RPT, RPT)],
                        out_hbm.at[c, pl.ds(s * RPT, RPT)])

    return segsum


def _make_counts(nchunk):
    @functools.partial(
        pl.kernel,
        mesh=plsc.VectorSubcoreMesh(core_axis_name="c", subcore_axis_name="s"),
        out_type=jax.ShapeDtypeStruct((NC, PADN, CW), jnp.float32),
        scratch_types=[
            pltpu.VMEM((nchunk, CH), jnp.int32),
            pltpu.VMEM((CH, CW), jnp.float32),
            pltpu.VMEM_SHARED((PADN, CW), jnp.float32),
        ],
    )
    def counts(dst_hbm, out_hbm, dst_v, ones_v, acc_sh):
        c = lax.axis_index("c")
        s = lax.axis_index("s")

        def fill(val):
            def body(i, carry):
                for k in range(CW // 16):
                    ones_v[i, pl.ds(k * 16, 16)] = jnp.full(
                        (16,), val, jnp.float32)
                return carry
            lax.fori_loop(0, CH, body, 0)

        fill(0.0)

        def zcp(t, carry):
            pltpu.sync_copy(ones_v, acc_sh.at[pl.ds(s * RPT + t * CH, CH)])
            return carry
        lax.fori_loop(0, RPT // CH, zcp, 0)

        fill(1.0)
        pltpu.sync_copy(dst_hbm.at[c, s], dst_v)
        plsc.subcore_barrier()

        def step(j, carry):
            pltpu.sync_copy(ones_v, acc_sh.at[dst_v.at[j]], add=True)
            return carry
        lax.fori_loop(0, nchunk, step, 0)

        plsc.subcore_barrier()
        pltpu.sync_copy(acc_sh.at[pl.ds(s * RPT, RPT)],
                        out_hbm.at[c, pl.ds(s * RPT, RPT)])

    return counts


def kernel(x, edge_index_follow, edge_index_friend,
           W_des, b_des, W_tweet, b_tweet, W_num, b_num, W_cat, b_cat,
           W_in, b_in, rgcn_weight, rgcn_root, rgcn_bias,
           W_out1, b_out1, W_out2, b_out2):
    E = edge_index_follow.shape[1]
    grp = NS * CH * 4
    ept = 4 * CH * ((E + grp - 1) // grp)
    nchunk = ept // CH
    pade = NS * ept

    des = x[:, 17:785]
    tweet = x[:, 785:1553]
    ncp = jnp.pad(x[:, 0:17], ((0, 0), (0, 15)))

    w_nc = jnp.zeros((32, 64), jnp.float32)
    w_nc = w_nc.at[0:6, 0:32].set(W_num).at[6:17, 32:64].set(W_cat)
    b_nc = jnp.concatenate([b_num, b_cat]).reshape(1, 64)

    pad_dst = N + (jnp.arange(pade - E, dtype=jnp.int32) % (PADN - N))
    pad_src = jnp.arange(pade - E, dtype=jnp.int32) % N

    def pad_edges(ei):
        src = jnp.concatenate([ei[0], pad_src])
        dst = jnp.concatenate([ei[1], pad_dst])
        return src, dst

    src_f, dst_f = pad_edges(edge_index_follow)
    src_r, dst_r = pad_edges(edge_index_friend)
    src_all = jnp.stack([src_f, src_r]).reshape(NC, NS, nchunk, CH)
    dst_all = jnp.stack([dst_f, dst_r]).reshape(NC, NS, nchunk, CH)
    eidx_all = (src_all << 16) | dst_all

    segsum = _make_segsum(nchunk)
    cnts = _make_counts(nchunk)(dst_all)

    h1 = _embed_call(
        des, tweet, ncp,
        W_des, W_tweet, w_nc,
        W_in[0:32], W_in[32:64], W_in[64:128],
        b_des.reshape(1, 32), b_tweet.reshape(1, 32), b_nc,
        b_in.reshape(1, EMB))

    seg1 = segsum(h1, eidx_all)
    h2 = _combine_call(h1, seg1, seg1, cnts, cnts,
                       rgcn_root, rgcn_weight[0], rgcn_weight[1],
                       rgcn_bias.reshape(1, EMB))

    seg2 = segsum(h2, eidx_all)
    out = _final_call(h2, seg2, seg2, cnts, cnts,
                      rgcn_root, rgcn_weight[0], rgcn_weight[1],
                      rgcn_bias.reshape(1, EMB),
                      W_out1, b_out1.reshape(1, EMB),
                      W_out2, b_out2.reshape(1, 2))
    return out


# speedup vs baseline: 2.5298x; 1.0889x over previous
import functools

import jax
import jax.numpy as jnp
from jax import lax
from jax.experimental import pallas as pl
from jax.experimental.pallas import tpu as pltpu
from jax.experimental.pallas import tpu_sc as plsc

N = 10000
EMB = 128
NBLK = 10
BLK = N // NBLK
NC = 2
NS = 16
CH = 128
PADN = 10240
RPT = PADN // NS
CW = 128


def _lk(v):
    return jnp.maximum(v, 0.01 * v)


def _dot(a, b):
    return jnp.dot(a, b, preferred_element_type=jnp.float32)


def _embed_body(x_ref, wbig, wi, bbig, bi, out_ref):
    h = _lk(_dot(x_ref[...], wbig[...]) + bbig[...])
    out_ref[...] = _lk(_dot(h, wi[...]) + bi[...])


def _combine_body(h_ref, s0_ref, s1_ref, c0_ref, c1_ref,
                  root, w0, w1, b, out_ref):
    m0 = s0_ref[0] / jnp.maximum(c0_ref[0][:, 0:1], 1.0)
    m1 = s1_ref[0] / jnp.maximum(c1_ref[0][:, 0:1], 1.0)
    out_ref[...] = (_dot(h_ref[...], root[...]) + _dot(m0, w0[...])
                    + _dot(m1, w1[...]) + b[...])


def _final_body(h_ref, s0_ref, s1_ref, c0_ref, c1_ref,
                root, w0, w1, b, wo1, bo1, wo2, bo2, out_ref):
    m0 = s0_ref[0] / jnp.maximum(c0_ref[0][:, 0:1], 1.0)
    m1 = s1_ref[0] / jnp.maximum(c1_ref[0][:, 0:1], 1.0)
    h3 = (_dot(h_ref[...], root[...]) + _dot(m0, w0[...])
          + _dot(m1, w1[...]) + b[...])
    g = _lk(_dot(h3, wo1[...]) + bo1[...])
    out_ref[...] = _dot(g, wo2[...]) + bo2[...]


def _full(shape):
    return pl.BlockSpec(shape, lambda i: tuple(0 for _ in shape))


def _rows(width):
    return pl.BlockSpec((BLK, width), lambda i: (i, 0))


def _seg_spec(r, width):
    return pl.BlockSpec((1, BLK, width), lambda i, _r=r: (_r, i, 0))


_embed_call = pl.pallas_call(
    _embed_body,
    grid=(NBLK,),
    in_specs=[
        _rows(1553),
        _full((1553, EMB)), _full((EMB, EMB)),
        _full((1, EMB)), _full((1, EMB)),
    ],
    out_specs=_rows(EMB),
    out_shape=jax.ShapeDtypeStruct((N, EMB), jnp.float32),
)

_combine_call = pl.pallas_call(
    _combine_body,
    grid=(NBLK,),
    in_specs=[
        _rows(EMB), _seg_spec(0, EMB), _seg_spec(1, EMB),
        _seg_spec(0, CW), _seg_spec(1, CW),
        _full((EMB, EMB)), _full((EMB, EMB)), _full((EMB, EMB)),
        _full((1, EMB)),
    ],
    out_specs=_rows(EMB),
    out_shape=jax.ShapeDtypeStruct((N, EMB), jnp.float32),
)

_final_call = pl.pallas_call(
    _final_body,
    grid=(NBLK,),
    in_specs=[
        _rows(EMB), _seg_spec(0, EMB), _seg_spec(1, EMB),
        _seg_spec(0, CW), _seg_spec(1, CW),
        _full((EMB, EMB)), _full((EMB, EMB)), _full((EMB, EMB)),
        _full((1, EMB)),
        _full((EMB, EMB)), _full((1, EMB)), _full((EMB, 2)), _full((1, 2)),
    ],
    out_specs=_rows(2),
    out_shape=jax.ShapeDtypeStruct((N, 2), jnp.float32),
)


def _make_segsum(nchunk):
    assert nchunk % 4 == 0 and nchunk >= 8

    @functools.partial(
        pl.kernel,
        mesh=plsc.VectorSubcoreMesh(core_axis_name="c", subcore_axis_name="s"),
        out_type=jax.ShapeDtypeStruct((NC, PADN, EMB), jnp.float32),
        scratch_types=[
            pltpu.VMEM((nchunk, CH), jnp.int32),
            pltpu.VMEM((2, 2, CH), jnp.int32),
            pltpu.VMEM((2, CH, EMB), jnp.float32),
            pltpu.VMEM_SHARED((PADN, EMB), jnp.float32),
            pltpu.SemaphoreType.DMA,
            pltpu.SemaphoreType.DMA,
        ],
    )
    def segsum(h_hbm, eidx_hbm, out_hbm,
               packed_v, ibuf, rows, acc_sh, sg0, sg1):
        c = lax.axis_index("c")
        s = lax.axis_index("s")
        sem_g = (sg0, sg1)

        def zrow(i, carry):
            for k in range(EMB // 16):
                rows[0, i, pl.ds(k * 16, 16)] = jnp.zeros((16,), jnp.float32)
            return carry
        lax.fori_loop(0, CH, zrow, 0)

        def zcp(t, carry):
            pltpu.sync_copy(rows.at[0],
                            acc_sh.at[pl.ds(s * RPT + t * CH, CH)])
            return carry
        lax.fori_loop(0, RPT // CH, zcp, 0)

        pltpu.sync_copy(eidx_hbm.at[c, s], packed_v)
        plsc.subcore_barrier()

        def unpack(j, p):
            def u(k, carry):
                v = packed_v[j, pl.ds(k * 16, 16)]
                ibuf[p, 0, pl.ds(k * 16, 16)] = lax.shift_right_logical(v, 16)
                ibuf[p, 1, pl.ds(k * 16, 16)] = lax.bitwise_and(v, 0xFFFF)
                return carry
            lax.fori_loop(0, CH // 16, u, 0)

        def gather(p):
            pltpu.async_copy(h_hbm.at[ibuf.at[p, 0]], rows.at[p], sem_g[p])

        def wait_gather(p):
            pltpu.make_async_copy(
                h_hbm.at[ibuf.at[p, 0]], rows.at[p], sem_g[p]).wait()

        def step(j, p, next_gather, unpack2, jnext):
            q = 1 - p
            if next_gather:
                gather(q)
            wait_gather(p)
            pltpu.sync_copy(rows.at[p], acc_sh.at[ibuf.at[p, 1]], add=True)
            if unpack2:
                unpack(jnext, p)

        unpack(0, 0)
        gather(0)
        unpack(1, 1)

        def body(i, carry):
            base = 4 * i
            for u in range(4):
                step(base + u, u % 2, True, True, base + u + 2)
            return carry
        lax.fori_loop(0, (nchunk - 4) // 4, body, 0)

        for j in range(nchunk - 4, nchunk):
            step(j, j % 2, j + 1 < nchunk, j + 2 < nchunk, j + 2)

        plsc.subcore_barrier()
        pltpu.sync_copy(acc_sh.at[pl.ds(s * RPT, RPT)],
                        out_hbm.at[c, pl.ds(s * RPT, RPT)])

    return segsum


def _make_counts(nchunk):
    @functools.partial(
        pl.kernel,
        mesh=plsc.VectorSubcoreMesh(core_axis_name="c", subcore_axis_name="s"),
        out_type=jax.ShapeDtypeStruct((NC, PADN, CW), jnp.float32),
        scratch_types=[
            pltpu.VMEM((nchunk, CH), jnp.int32),
            pltpu.VMEM((CH, CW), jnp.float32),
            pltpu.VMEM_SHARED((PADN, CW), jnp.float32),
        ],
    )
    def counts(dst_hbm, out_hbm, dst_v, ones_v, acc_sh):
        c = lax.axis_index("c")
        s = lax.axis_index("s")

        def fill(val):
            def body(i, carry):
                for k in range(CW // 16):
                    ones_v[i, pl.ds(k * 16, 16)] = jnp.full(
                        (16,), val, jnp.float32)
                return carry
            lax.fori_loop(0, CH, body, 0)

        fill(0.0)

        def zcp(t, carry):
            pltpu.sync_copy(ones_v, acc_sh.at[pl.ds(s * RPT + t * CH, CH)])
            return carry
        lax.fori_loop(0, RPT // CH, zcp, 0)

        fill(1.0)
        pltpu.sync_copy(dst_hbm.at[c, s], dst_v)
        plsc.subcore_barrier()

        def step(j, carry):
            pltpu.sync_copy(ones_v, acc_sh.at[dst_v.at[j]], add=True)
            return carry
        lax.fori_loop(0, nchunk, step, 0)

        plsc.subcore_barrier()
        pltpu.sync_copy(acc_sh.at[pl.ds(s * RPT, RPT)],
                        out_hbm.at[c, pl.ds(s * RPT, RPT)])

    return counts


def kernel(x, edge_index_follow, edge_index_friend,
           W_des, b_des, W_tweet, b_tweet, W_num, b_num, W_cat, b_cat,
           W_in, b_in, rgcn_weight, rgcn_root, rgcn_bias,
           W_out1, b_out1, W_out2, b_out2):
    E = edge_index_follow.shape[1]
    grp = NS * CH * 4
    ept = 4 * CH * ((E + grp - 1) // grp)
    nchunk = ept // CH
    pade = NS * ept

    w_big = jnp.zeros((1553, EMB), jnp.float32)
    w_big = (w_big.at[17:785, 0:32].set(W_des)
             .at[785:1553, 32:64].set(W_tweet)
             .at[0:6, 64:96].set(W_num)
             .at[6:17, 96:128].set(W_cat))
    b_big = jnp.concatenate([b_des, b_tweet, b_num, b_cat]).reshape(1, EMB)

    pad_dst = N + (jnp.arange(pade - E, dtype=jnp.int32) % (PADN - N))
    pad_src = jnp.arange(pade - E, dtype=jnp.int32) % N

    def pad_edges(ei):
        src = jnp.concatenate([ei[0], pad_src])
        dst = jnp.concatenate([ei[1], pad_dst])
        return src, dst

    src_f, dst_f = pad_edges(edge_index_follow)
    src_r, dst_r = pad_edges(edge_index_friend)
    src_all = jnp.stack([src_f, src_r]).reshape(NC, NS, nchunk, CH)
    dst_all = jnp.stack([dst_f, dst_r]).reshape(NC, NS, nchunk, CH)
    eidx_all = (src_all << 16) | dst_all

    segsum = _make_segsum(nchunk)
    cnts = _make_counts(nchunk)(dst_all)

    h1 = _embed_call(x, w_big, W_in, b_big, b_in.reshape(1, EMB))

    seg1 = segsum(h1, eidx_all)
    h2 = _combine_call(h1, seg1, seg1, cnts, cnts,
                       rgcn_root, rgcn_weight[0], rgcn_weight[1],
                       rgcn_bias.reshape(1, EMB))

    seg2 = segsum(h2, eidx_all)
    out = _final_call(h2, seg2, seg2, cnts, cnts,
                      rgcn_root, rgcn_weight[0], rgcn_weight[1],
                      rgcn_bias.reshape(1, EMB),
                      W_out1, b_out1.reshape(1, EMB),
                      W_out2, b_out2.reshape(1, 2))
    return out
